# baseline (speedup 1.0000x reference)
import jax
import jax.numpy as jnp
from jax import lax
from jax.experimental import pallas as pl
from jax.experimental.pallas import tpu as pltpu
from jax.experimental.pallas import tpu_sc as plsc

D = 64
B = 16384
NC, NS = 2, 16
NW = NC * NS
BPW = B // NW
L = 16
CH = 128
NCH = BPW // CH
W2 = 2 * D


def _body(head_hbm, rel_hbm, tail_hbm, ent_hbm, relemb_hbm, out_hbm,
          hidx_v, ridx_v, tidx_v, hrow_v, rrow_v, trow_v,
          hbuf_v, rbuf_v, tbuf_v, q_v, out_v, sems):
    wid = lax.axis_index("s") * NC + lax.axis_index("c")
    base = wid * BPW

    pltpu.sync_copy(head_hbm.at[pl.ds(base, BPW)], hidx_v)
    pltpu.sync_copy(rel_hbm.at[pl.ds(base, BPW)], ridx_v)
    pltpu.sync_copy(tail_hbm.at[pl.ds(base, BPW)], tidx_v)

    for v in range(BPW // L):
        hrow_v[pl.ds(v * L, L)] = jnp.right_shift(hidx_v[pl.ds(v * L, L)], 1)
        rrow_v[pl.ds(v * L, L)] = jnp.right_shift(ridx_v[pl.ds(v * L, L)], 1)
        trow_v[pl.ds(v * L, L)] = jnp.right_shift(tidx_v[pl.ds(v * L, L)], 1)

    def issue(k):
        slot = lax.rem(k, 2)
        o = slot * CH
        pltpu.async_copy(ent_hbm.at[hrow_v.at[pl.ds(k * CH, CH)]],
                         hbuf_v.at[pl.ds(o, CH)], sems.at[slot])
        pltpu.async_copy(relemb_hbm.at[rrow_v.at[pl.ds(k * CH, CH)]],
                         rbuf_v.at[pl.ds(o, CH)], sems.at[slot])
        pltpu.async_copy(ent_hbm.at[trow_v.at[pl.ds(k * CH, CH)]],
                         tbuf_v.at[pl.ds(o, CH)], sems.at[slot])

    def drain(k):
        slot = lax.rem(k, 2)
        o = slot * CH
        pltpu.make_async_copy(ent_hbm.at[hrow_v.at[pl.ds(0, CH)]],
                              hbuf_v.at[pl.ds(o, CH)], sems.at[slot]).wait()
        pltpu.make_async_copy(ent_hbm.at[hrow_v.at[pl.ds(0, CH)]],
                              rbuf_v.at[pl.ds(o, CH)], sems.at[slot]).wait()
        pltpu.make_async_copy(ent_hbm.at[hrow_v.at[pl.ds(0, CH)]],
                              tbuf_v.at[pl.ds(o, CH)], sems.at[slot]).wait()

    issue(0)

    def chunk(k, carry):
        drain(k)

        @pl.when(k + 1 < NCH)
        def _():
            issue(k + 1)

        slot = lax.rem(k, 2)
        o = slot * CH

        def grp(gg, carry2):
            b0 = k * CH + gg * L
            vh = jnp.bitwise_and(hidx_v[pl.ds(b0, L)], 1) * D
            vr = jnp.bitwise_and(ridx_v[pl.ds(b0, L)], 1) * D
            vt = jnp.bitwise_and(tidx_v[pl.ds(b0, L)], 1) * D
            for j in range(L):
                i = gg * L + j
                hh, rh, th = vh[j], vr[j], vt[j]
                q = (hbuf_v[o + i, pl.ds(hh, L)] * rbuf_v[o + i, pl.ds(rh, L)]
                     * tbuf_v[o + i, pl.ds(th, L)])
                for c in range(L, D, L):
                    q += (hbuf_v[o + i, pl.ds(hh + c, L)]
                          * rbuf_v[o + i, pl.ds(rh + c, L)]
                          * tbuf_v[o + i, pl.ds(th + c, L)])
                q_v[pl.ds((b0 + j) * L, L)] = q
            return carry2

        lax.fori_loop(0, CH // L, grp, 0)
        return carry

    lax.fori_loop(0, NCH, chunk, 0)

    @plsc.parallel_loop(0, BPW // L, 1, unroll=2)
    def _rowB(g):
        rows = g * L + jnp.arange(L, dtype=jnp.int32)
        acc = plsc.load_gather(q_v, [rows * L])
        for l in range(1, L):
            acc += plsc.load_gather(q_v, [rows * L + l])
        out_v[pl.ds(g * L, L)] = acc

    pltpu.sync_copy(out_v, out_hbm.at[pl.ds(base, BPW)])


@jax.jit
def _distmult(head, relation, tail, ent2, rel2):
    mesh = plsc.VectorSubcoreMesh(core_axis_name="c", subcore_axis_name="s")
    return pl.kernel(
        _body,
        out_type=jax.ShapeDtypeStruct((B,), jnp.float32),
        mesh=mesh,
        scratch_types=[
            pltpu.VMEM((BPW,), jnp.int32),
            pltpu.VMEM((BPW,), jnp.int32),
            pltpu.VMEM((BPW,), jnp.int32),
            pltpu.VMEM((BPW,), jnp.int32),
            pltpu.VMEM((BPW,), jnp.int32),
            pltpu.VMEM((BPW,), jnp.int32),
            pltpu.VMEM((2 * CH, W2), jnp.float32),
            pltpu.VMEM((2 * CH, W2), jnp.float32),
            pltpu.VMEM((2 * CH, W2), jnp.float32),
            pltpu.VMEM((BPW * L,), jnp.float32),
            pltpu.VMEM((BPW,), jnp.float32),
            pltpu.SemaphoreType.DMA((2,)),
        ],
        compiler_params=pltpu.CompilerParams(
            needs_layout_passes=False, use_tc_tiling_on_sc=True),
    )(head, relation, tail, ent2, rel2)


def kernel(head, relation, tail, entity_emb, relation_emb):
    ent2 = entity_emb.reshape(entity_emb.shape[0] // 2, W2)
    rel2 = relation_emb.reshape(relation_emb.shape[0] // 2, W2)
    return _distmult(head.astype(jnp.int32), relation.astype(jnp.int32),
                     tail.astype(jnp.int32), ent2, rel2)


# speedup vs baseline: 1.1201x; 1.1201x over previous
import jax
import jax.numpy as jnp
from jax import lax
from jax.experimental import pallas as pl
from jax.experimental.pallas import tpu as pltpu
from jax.experimental.pallas import tpu_sc as plsc

D = 64
B = 16384
NC, NS = 2, 16
NW = NC * NS
BPW = B // NW
L = 16
CH = 128
NCH = BPW // CH
W2 = 2 * D


def _body(head_hbm, rel_hbm, tail_hbm, ent_hbm, relemb_hbm, out_hbm,
          hidx_v, ridx_v, tidx_v,
          hbuf_v, rbuf_v, tbuf_v, q_v, out_v, sems):
    wid = lax.axis_index("s") * NC + lax.axis_index("c")
    base = wid * BPW

    pltpu.sync_copy(head_hbm.at[pl.ds(base, BPW)], hidx_v)
    pltpu.sync_copy(rel_hbm.at[pl.ds(base, BPW)], ridx_v)
    pltpu.sync_copy(tail_hbm.at[pl.ds(base, BPW)], tidx_v)


    def issue(k):
        slot = lax.rem(k, 2)
        o = slot * CH
        pltpu.async_copy(ent_hbm.at[hidx_v.at[pl.ds(k * CH, CH)]],
                         hbuf_v.at[pl.ds(o, CH)], sems.at[slot])
        pltpu.async_copy(relemb_hbm.at[ridx_v.at[pl.ds(k * CH, CH)]],
                         rbuf_v.at[pl.ds(o, CH)], sems.at[slot])
        pltpu.async_copy(ent_hbm.at[tidx_v.at[pl.ds(k * CH, CH)]],
                         tbuf_v.at[pl.ds(o, CH)], sems.at[slot])

    def drain(k):
        slot = lax.rem(k, 2)
        o = slot * CH
        pltpu.make_async_copy(ent_hbm.at[hidx_v.at[pl.ds(0, CH)]],
                              hbuf_v.at[pl.ds(o, CH)], sems.at[slot]).wait()
        pltpu.make_async_copy(ent_hbm.at[hidx_v.at[pl.ds(0, CH)]],
                              rbuf_v.at[pl.ds(o, CH)], sems.at[slot]).wait()
        pltpu.make_async_copy(ent_hbm.at[hidx_v.at[pl.ds(0, CH)]],
                              tbuf_v.at[pl.ds(o, CH)], sems.at[slot]).wait()

    issue(0)

    def chunk(k, carry):
        drain(k)

        @pl.when(k + 1 < NCH)
        def _():
            issue(k + 1)

        slot = lax.rem(k, 2)
        o = slot * CH

        def row(i, carry2):
            b = k * CH + i
            q = (hbuf_v[o + i, pl.ds(0, L)] * rbuf_v[o + i, pl.ds(0, L)]
                 * tbuf_v[o + i, pl.ds(0, L)])
            for c in range(L, D, L):
                q += (hbuf_v[o + i, pl.ds(c, L)] * rbuf_v[o + i, pl.ds(c, L)]
                      * tbuf_v[o + i, pl.ds(c, L)])
            q_v[pl.ds(b * L, L)] = q
            return carry2

        lax.fori_loop(0, CH, row, 0)
        return carry

    lax.fori_loop(0, NCH, chunk, 0)

    @plsc.parallel_loop(0, BPW // L, 1, unroll=2)
    def _rowB(g):
        rows = g * L + jnp.arange(L, dtype=jnp.int32)
        acc = plsc.load_gather(q_v, [rows * L])
        for l in range(1, L):
            acc += plsc.load_gather(q_v, [rows * L + l])
        out_v[pl.ds(g * L, L)] = acc

    pltpu.sync_copy(out_v, out_hbm.at[pl.ds(base, BPW)])


@jax.jit
def _distmult(head, relation, tail, ent2, rel2):
    mesh = plsc.VectorSubcoreMesh(core_axis_name="c", subcore_axis_name="s")
    return pl.kernel(
        _body,
        out_type=jax.ShapeDtypeStruct((B,), jnp.float32),
        mesh=mesh,
        scratch_types=[
            pltpu.VMEM((BPW,), jnp.int32),
            pltpu.VMEM((BPW,), jnp.int32),
            pltpu.VMEM((BPW,), jnp.int32),
            pltpu.VMEM((2 * CH, W2), jnp.float32),
            pltpu.VMEM((2 * CH, W2), jnp.float32),
            pltpu.VMEM((2 * CH, W2), jnp.float32),
            pltpu.VMEM((BPW * L,), jnp.float32),
            pltpu.VMEM((BPW,), jnp.float32),
            pltpu.SemaphoreType.DMA((2,)),
        ],
        compiler_params=pltpu.CompilerParams(
            needs_layout_passes=False, use_tc_tiling_on_sc=True),
    )(head, relation, tail, ent2, rel2)


def kernel(head, relation, tail, entity_emb, relation_emb):
    ent2 = jnp.pad(entity_emb, ((0, 0), (0, W2 - D)))
    rel2 = jnp.pad(relation_emb, ((0, 0), (0, W2 - D)))
    return _distmult(head.astype(jnp.int32), relation.astype(jnp.int32),
                     tail.astype(jnp.int32), ent2, rel2)
